# baseline (speedup 1.0000x reference)
import jax
import jax.numpy as jnp
from jax.experimental import pallas as pl
from jax.experimental.pallas import tpu as pltpu

_R = 16
_E = 64
_B = 4
_DIN = 1024
_DOUT = 1024
_SEQ = 2048
_SBLK = 1024
_NH = _SEQ // _SBLK
_NEG_INF = float("-inf")


def _topk_onehots(scores):
    col = jax.lax.broadcasted_iota(jnp.int32, (_B, _E), 1)
    run = scores
    ohs = []
    for _ in range(_R):
        m = jnp.max(run, axis=1, keepdims=True)
        cand = jnp.where(run == m, col, _E)
        amin = jnp.min(cand, axis=1, keepdims=True)
        oh = col == amin
        ohs.append(oh.astype(jnp.float32))
        run = jnp.where(oh, _NEG_INF, run)
    return ohs


def _fused_kernel(q_ref, wa_ref, ba_ref, wb_ref, bb_ref,
                  at_ref, cfs_ref, bt_ref, x_ref, out_ref,
                  la_s, bs_s, ohb_s):
    i = pl.program_id(0)

    @pl.when(i == 0)
    def _route():
        q = q_ref[...]
        s_a = jax.lax.dot_general(q, wa_ref[...], (((1,), (1,)), ((), ())),
                                  preferred_element_type=jnp.float32)
        oh_a = _topk_onehots(s_a + ba_ref[...])
        s_b = jax.lax.dot_general(q, wb_ref[...], (((1,), (1,)), ((), ())),
                                  preferred_element_type=jnp.float32)
        s_b = s_b + bb_ref[...]
        for r in range(_R):
            la_r = jnp.dot(oh_a[r], at_ref[:, r, :],
                           preferred_element_type=jnp.float32)
            la_s[:, r, :] = la_r
            s_b = s_b + jax.lax.dot_general(
                la_r, cfs_ref[r], (((1,), (1,)), ((), ())),
                preferred_element_type=jnp.float32)
        oh_b = _topk_onehots(s_b)
        for r in range(_R):
            ohb_s[r] = oh_b[r]

    for b in range(_B):
        for h in range(_NH):
            @pl.when(i == 1 + b * _NH + h)
            def _apply(b=b, h=h):
                if h == 0:
                    for r in range(_R):
                        bs_s[b, r, :] = jnp.dot(
                            ohb_s[r, b:b + 1, :], bt_ref[r],
                            preferred_element_type=jnp.float32)[0]
                x = x_ref[0]
                after = jax.lax.dot_general(
                    x, la_s[b], (((1,), (1,)), ((), ())),
                    preferred_element_type=jnp.float32)
                out_ref[0] = jnp.dot(after, bs_s[b],
                                     preferred_element_type=jnp.float32)


def _run(x, query_signal, A_pool, B_pool, W_A, b_A, W_B, b_B, cfs_W,
         interpret=False):
    n_exp = A_pool.shape[0]
    bt = jnp.transpose(B_pool, (2, 0, 1))
    cfst = jnp.transpose(cfs_W, (0, 2, 1))
    out = pl.pallas_call(
        _fused_kernel,
        grid=(1 + _B * _NH,),
        in_specs=[
            pl.BlockSpec((_B, _DIN), lambda i: (0, 0)),
            pl.BlockSpec((_E, _DIN), lambda i: (0, 0)),
            pl.BlockSpec((1, _E), lambda i: (0, 0)),
            pl.BlockSpec((_E, _DIN), lambda i: (0, 0)),
            pl.BlockSpec((1, _E), lambda i: (0, 0)),
            pl.BlockSpec((_E, _R, _DIN), lambda i: (0, 0, 0)),
            pl.BlockSpec((_R, _E, _DIN), lambda i: (0, 0, 0)),
            pl.BlockSpec((_R, _E, _DOUT), lambda i: (0, 0, 0)),
            pl.BlockSpec((1, _SBLK, _DIN),
                         lambda i: (jnp.maximum(i - 1, 0) // _NH,
                                    jnp.maximum(i - 1, 0) % _NH, 0)),
        ],
        out_specs=pl.BlockSpec(
            (1, _SBLK, _DOUT),
            lambda i: (jnp.maximum(i - 1, 0) // _NH,
                       jnp.maximum(i - 1, 0) % _NH, 0)),
        out_shape=jax.ShapeDtypeStruct((_B, _SEQ, _DOUT), jnp.float32),
        scratch_shapes=[
            pltpu.VMEM((_B, _R, _DIN), jnp.float32),
            pltpu.VMEM((_B, _R, _DOUT), jnp.float32),
            pltpu.VMEM((_R, _B, _E), jnp.float32),
        ],
        compiler_params=pltpu.CompilerParams(
            dimension_semantics=("arbitrary",)),
        interpret=interpret,
    )(query_signal, W_A, b_A.reshape(1, n_exp), W_B, b_B.reshape(1, n_exp),
      A_pool, cfst, bt, x)
    return out


def kernel(x, query_signal, A_pool, B_pool, W_A, b_A, W_B, b_B, cfs_W):
    return _run(x, query_signal, A_pool, B_pool, W_A, b_A, W_B, b_B, cfs_W)


# speedup vs baseline: 1.0165x; 1.0165x over previous
import jax
import jax.numpy as jnp
from jax.experimental import pallas as pl
from jax.experimental.pallas import tpu as pltpu

_R = 16
_E = 64
_B = 4
_DIN = 1024
_DOUT = 1024
_SEQ = 2048
_SBLK = 1024
_NH = _SEQ // _SBLK
_NEG_INF = float("-inf")


def _topk_onehots(scores):
    col = jax.lax.broadcasted_iota(jnp.int32, (_B, _E), 1)
    run = scores
    ohs = []
    for _ in range(_R):
        m = jnp.max(run, axis=1, keepdims=True)
        cand = jnp.where(run == m, col, _E)
        amin = jnp.min(cand, axis=1, keepdims=True)
        oh = col == amin
        ohs.append(oh.astype(jnp.float32))
        run = jnp.where(oh, _NEG_INF, run)
    return ohs


def _fused_kernel(q_ref, wa_ref, ba_ref, wb_ref, bb_ref,
                  at_ref, cfs_ref, bt_ref, x_ref, out_ref,
                  la_s, bs_s):
    i = pl.program_id(0)

    @pl.when(i == 0)
    def _route():
        q = q_ref[...]
        s_a = jax.lax.dot_general(q, wa_ref[...], (((1,), (1,)), ((), ())),
                                  preferred_element_type=jnp.float32)
        oh_a = _topk_onehots(s_a + ba_ref[...])
        s_b = jax.lax.dot_general(q, wb_ref[...], (((1,), (1,)), ((), ())),
                                  preferred_element_type=jnp.float32)
        s_b = s_b + bb_ref[...]
        for r in range(_R):
            la_r = jnp.dot(oh_a[r], at_ref[:, r, :],
                           preferred_element_type=jnp.float32)
            la_s[:, r, :] = la_r
            s_b = s_b + jax.lax.dot_general(
                la_r, cfs_ref[r], (((1,), (1,)), ((), ())),
                preferred_element_type=jnp.float32)
        oh_b = _topk_onehots(s_b)
        for r in range(_R):
            bs_s[:, r, :] = jnp.dot(oh_b[r], bt_ref[r],
                                    preferred_element_type=jnp.float32)

    for b in range(_B):
        for h in range(_NH):
            @pl.when(i == 1 + b * _NH + h)
            def _apply(b=b):
                x = x_ref[0]
                after = jax.lax.dot_general(
                    x, la_s[b], (((1,), (1,)), ((), ())),
                    preferred_element_type=jnp.float32)
                out_ref[0] = jnp.dot(after, bs_s[b],
                                     preferred_element_type=jnp.float32)


def _run(x, query_signal, A_pool, B_pool, W_A, b_A, W_B, b_B, cfs_W,
         interpret=False):
    n_exp = A_pool.shape[0]
    bt = jnp.transpose(B_pool, (2, 0, 1))
    cfst = jnp.transpose(cfs_W, (0, 2, 1))
    out = pl.pallas_call(
        _fused_kernel,
        grid=(1 + _B * _NH,),
        in_specs=[
            pl.BlockSpec((_B, _DIN), lambda i: (0, 0)),
            pl.BlockSpec((_E, _DIN), lambda i: (0, 0)),
            pl.BlockSpec((1, _E), lambda i: (0, 0)),
            pl.BlockSpec((_E, _DIN), lambda i: (0, 0)),
            pl.BlockSpec((1, _E), lambda i: (0, 0)),
            pl.BlockSpec((_E, _R, _DIN), lambda i: (0, 0, 0)),
            pl.BlockSpec((_R, _E, _DIN), lambda i: (0, 0, 0)),
            pl.BlockSpec((_R, _E, _DOUT), lambda i: (0, 0, 0)),
            pl.BlockSpec((1, _SBLK, _DIN),
                         lambda i: (jnp.maximum(i - 1, 0) // _NH,
                                    jnp.maximum(i - 1, 0) % _NH, 0)),
        ],
        out_specs=pl.BlockSpec(
            (1, _SBLK, _DOUT),
            lambda i: (jnp.maximum(i - 1, 0) // _NH,
                       jnp.maximum(i - 1, 0) % _NH, 0)),
        out_shape=jax.ShapeDtypeStruct((_B, _SEQ, _DOUT), jnp.float32),
        scratch_shapes=[
            pltpu.VMEM((_B, _R, _DIN), jnp.float32),
            pltpu.VMEM((_B, _R, _DOUT), jnp.float32),
        ],
        compiler_params=pltpu.CompilerParams(
            dimension_semantics=("arbitrary",)),
        interpret=interpret,
    )(query_signal, W_A, b_A.reshape(1, n_exp), W_B, b_B.reshape(1, n_exp),
      A_pool, cfst, bt, x)
    return out


def kernel(x, query_signal, A_pool, B_pool, W_A, b_A, W_B, b_B, cfs_W):
    return _run(x, query_signal, A_pool, B_pool, W_A, b_A, W_B, b_B, cfs_W)
